# baseline (speedup 1.0000x reference)
import jax
import jax.numpy as jnp
from jax import lax
from jax.experimental import pallas as pl
from jax.experimental.pallas import tpu as pltpu

EMBED_DIM = 2048
NUM_EXPERTS = 16
TOP_K = 2

BLOCK_T = 512
NSPLIT = 8
SUB_T = BLOCK_T // NSPLIT


def _router_block(x_hbm, w_ref, lt_ref, pk_ref, pw_ref, x_buf, sems):
    i = pl.program_id(0)
    nsteps = pl.num_programs(0)

    def copy(step, slot, s):
        return pltpu.make_async_copy(
            x_hbm.at[pl.ds(step * BLOCK_T + s * SUB_T, SUB_T), :],
            x_buf.at[slot, pl.ds(s * SUB_T, SUB_T), :],
            sems.at[slot, s],
        )

    slot = lax.rem(i, 2)
    nxt = lax.rem(i + 1, 2)

    @pl.when(i == 0)
    def _first():
        for s in range(NSPLIT):
            copy(0, 0, s).start()

    @pl.when(i + 1 < nsteps)
    def _prefetch():
        for s in range(NSPLIT):
            copy(i + 1, nxt, s).start()

    for s in range(NSPLIT):
        copy(i, slot, s).wait()

    lt = jax.lax.dot_general(
        w_ref[...], x_buf[slot],
        dimension_numbers=(((1,), (1,)), ((), ())),
        preferred_element_type=jnp.float32,
    )
    lt_ref[...] = lt

    iota = lax.broadcasted_iota(jnp.int32, lt.shape, 0)
    m1 = jnp.max(lt, axis=0, keepdims=True)
    i1 = jnp.min(jnp.where(lt == m1, iota, NUM_EXPERTS), axis=0,
                 keepdims=True)
    masked = jnp.where(iota == i1, -jnp.inf, lt)
    m2 = jnp.max(masked, axis=0, keepdims=True)
    i2 = jnp.min(jnp.where(masked == m2, iota, NUM_EXPERTS), axis=0,
                 keepdims=True)
    w1 = jax.nn.sigmoid(m1 - m2)
    pk_ref[...] = jnp.concatenate([i1, i2], axis=0)
    pw_ref[...] = jnp.concatenate([w1, 1.0 - w1], axis=0)


def kernel(x, W):
    n_tokens = x.shape[0]
    lt, pk, pw = pl.pallas_call(
        _router_block,
        grid=(n_tokens // BLOCK_T,),
        in_specs=[
            pl.BlockSpec(memory_space=pl.ANY),
            pl.BlockSpec((NUM_EXPERTS, EMBED_DIM), lambda i: (0, 0)),
        ],
        out_specs=(
            pl.BlockSpec((NUM_EXPERTS, BLOCK_T), lambda i: (0, i)),
            pl.BlockSpec((TOP_K, BLOCK_T), lambda i: (0, i)),
            pl.BlockSpec((TOP_K, BLOCK_T), lambda i: (0, i)),
        ),
        out_shape=(
            jax.ShapeDtypeStruct((NUM_EXPERTS, n_tokens), jnp.float32),
            jax.ShapeDtypeStruct((TOP_K, n_tokens), jnp.int32),
            jax.ShapeDtypeStruct((TOP_K, n_tokens), jnp.float32),
        ),
        scratch_shapes=[
            pltpu.VMEM((2, BLOCK_T, EMBED_DIM), jnp.float32),
            pltpu.SemaphoreType.DMA((2, NSPLIT)),
        ],
    )(x, W)
    return (pk.T, pw.T, lt.T)


# speedup vs baseline: 1.1939x; 1.1939x over previous
import jax
import jax.numpy as jnp
from jax import lax
from jax.experimental import pallas as pl
from jax.experimental.pallas import tpu as pltpu

EMBED_DIM = 2048
NUM_EXPERTS = 16
TOP_K = 2

BLOCK_T = 1024
NSPLIT = 4
SUB_T = BLOCK_T // NSPLIT


def _router_block(x_hbm, w_ref, lt_ref, pk_ref, pw_ref, x_buf, sems):
    i = pl.program_id(0)
    nsteps = pl.num_programs(0)

    def copy(step, slot, s):
        return pltpu.make_async_copy(
            x_hbm.at[pl.ds(step * BLOCK_T + s * SUB_T, SUB_T), :],
            x_buf.at[slot, pl.ds(s * SUB_T, SUB_T), :],
            sems.at[slot, s],
        )

    slot = lax.rem(i, 2)
    nxt = lax.rem(i + 1, 2)

    @pl.when(i == 0)
    def _first():
        for s in range(NSPLIT):
            copy(0, 0, s).start()

    @pl.when(i + 1 < nsteps)
    def _prefetch():
        for s in range(NSPLIT):
            copy(i + 1, nxt, s).start()

    for s in range(NSPLIT):
        copy(i, slot, s).wait()

    lt = jax.lax.dot_general(
        w_ref[...], x_buf[slot],
        dimension_numbers=(((1,), (1,)), ((), ())),
        preferred_element_type=jnp.float32,
    )
    lt_ref[...] = lt

    iota = lax.broadcasted_iota(jnp.int32, lt.shape, 0)
    m1 = jnp.max(lt, axis=0, keepdims=True)
    i1 = jnp.min(jnp.where(lt == m1, iota, NUM_EXPERTS), axis=0,
                 keepdims=True)
    masked = jnp.where(iota == i1, -jnp.inf, lt)
    m2 = jnp.max(masked, axis=0, keepdims=True)
    i2 = jnp.min(jnp.where(masked == m2, iota, NUM_EXPERTS), axis=0,
                 keepdims=True)
    w1 = jax.nn.sigmoid(m1 - m2)
    pk_ref[...] = jnp.concatenate([i1, i2], axis=0)
    pw_ref[...] = jnp.concatenate([w1, 1.0 - w1], axis=0)


def kernel(x, W):
    n_tokens = x.shape[0]
    lt, pk, pw = pl.pallas_call(
        _router_block,
        grid=(n_tokens // BLOCK_T,),
        in_specs=[
            pl.BlockSpec(memory_space=pl.ANY),
            pl.BlockSpec((NUM_EXPERTS, EMBED_DIM), lambda i: (0, 0)),
        ],
        out_specs=(
            pl.BlockSpec((NUM_EXPERTS, BLOCK_T), lambda i: (0, i)),
            pl.BlockSpec((TOP_K, BLOCK_T), lambda i: (0, i)),
            pl.BlockSpec((TOP_K, BLOCK_T), lambda i: (0, i)),
        ),
        out_shape=(
            jax.ShapeDtypeStruct((NUM_EXPERTS, n_tokens), jnp.float32),
            jax.ShapeDtypeStruct((TOP_K, n_tokens), jnp.int32),
            jax.ShapeDtypeStruct((TOP_K, n_tokens), jnp.float32),
        ),
        scratch_shapes=[
            pltpu.VMEM((2, BLOCK_T, EMBED_DIM), jnp.float32),
            pltpu.SemaphoreType.DMA((2, NSPLIT)),
        ],
    )(x, W)
    return (pk.T, pw.T, lt.T)
